# baseline (speedup 1.0000x reference)
import functools

import jax
import jax.numpy as jnp
from jax import lax
from jax.experimental import pallas as pl
from jax.experimental.pallas import tpu as pltpu
from jax.experimental.pallas import tpu_sc as plsc

B, C, H, W = 256, 256, 14, 14
HW = H * W


def _sc_gather_masks(indices, mask_table):
    info = plsc.get_sparse_core_info()
    nw = info.num_cores * info.num_subcores
    b_per_w = B // nw

    mesh = plsc.VectorSubcoreMesh(core_axis_name="c", subcore_axis_name="s")

    @functools.partial(
        pl.kernel,
        out_type=jax.ShapeDtypeStruct((B, C), jnp.bool_),
        mesh=mesh,
        scratch_types=[
            pltpu.VMEM((b_per_w,), jnp.int32),
            pltpu.VMEM((b_per_w, C), jnp.bool_),
            pltpu.SemaphoreType.DMA,
        ],
    )
    def gather_kernel(idx_hbm, table_hbm, out_hbm, idx_v, rows_v, sem):
        wid = lax.axis_index("s") * info.num_cores + lax.axis_index("c")
        base = wid * b_per_w
        pltpu.sync_copy(idx_hbm.at[pl.ds(base, b_per_w)], idx_v)
        pltpu.async_copy(table_hbm.at[idx_v], rows_v, sem).wait()
        pltpu.sync_copy(rows_v, out_hbm.at[pl.ds(base, b_per_w)])

    return gather_kernel(indices, mask_table)


def _tc_mask_mul(x_rows, mask_rows):
    rows_per_block = 2048
    grid = (B * C // rows_per_block,)

    def body(x_ref, m_ref, o_ref):
        o_ref[...] = x_ref[...] * m_ref[...].astype(jnp.float32)

    return pl.pallas_call(
        body,
        grid=grid,
        in_specs=[
            pl.BlockSpec((rows_per_block, HW), lambda i: (i, 0)),
            pl.BlockSpec((rows_per_block, 1), lambda i: (i, 0)),
        ],
        out_specs=pl.BlockSpec((rows_per_block, HW), lambda i: (i, 0)),
        out_shape=jax.ShapeDtypeStruct((B * C, HW), jnp.float32),
    )(x_rows, mask_rows)


def kernel(X, indices, mask_table):
    masks = _sc_gather_masks(indices, mask_table)
    out = _tc_mask_mul(X.reshape(B * C, HW), masks.reshape(B * C, 1))
    return out.reshape(B, C, H, W)


# speedup vs baseline: 2.1388x; 2.1388x over previous
import functools

import jax
import jax.numpy as jnp
from jax.experimental import pallas as pl
from jax.experimental.pallas import tpu as pltpu

B, C, H, W = 256, 256, 14, 14
HW = H * W


def _gather_mask_plane(indices, mask_table):
    def body(idx_ref, table_ref, o_ref):
        i = pl.program_id(0)
        r = idx_ref[i] % 8
        row = table_ref[pl.ds(r, 1), :].astype(jnp.float32)
        o_ref[pl.ds(i % 8, 1), :] = row

    grid_spec = pltpu.PrefetchScalarGridSpec(
        num_scalar_prefetch=1,
        grid=(B,),
        in_specs=[
            pl.BlockSpec((8, C), lambda i, idx: (idx[i] // 8, 0)),
        ],
        out_specs=pl.BlockSpec((8, C), lambda i, idx: (i // 8, 0)),
    )
    return pl.pallas_call(
        body,
        grid_spec=grid_spec,
        out_shape=jax.ShapeDtypeStruct((B, C), jnp.float32),
    )(indices, mask_table)


def _mask_multiply(x_planes, mask):
    p = 14
    grid = (HW // p,)

    def body(x_ref, m_ref, o_ref):
        o_ref[...] = x_ref[...] * m_ref[...][None, :, :]

    return pl.pallas_call(
        body,
        grid=grid,
        in_specs=[
            pl.BlockSpec((p, B, C), lambda i: (i, 0, 0)),
            pl.BlockSpec((B, C), lambda i: (0, 0)),
        ],
        out_specs=pl.BlockSpec((p, B, C), lambda i: (i, 0, 0)),
        out_shape=jax.ShapeDtypeStruct((HW, B, C), jnp.float32),
    )(x_planes, mask)


def kernel(X, indices, mask_table):
    mask = _gather_mask_plane(indices, mask_table)
    x_planes = jnp.transpose(X, (2, 3, 0, 1)).reshape(HW, B, C)
    out = _mask_multiply(x_planes, mask)
    return jnp.transpose(out.reshape(H, W, B, C), (2, 3, 0, 1))


# speedup vs baseline: 5.2206x; 2.4409x over previous
import jax
import jax.numpy as jnp
from jax.experimental import pallas as pl
from jax.experimental.pallas import tpu as pltpu

B, C, H, W = 256, 256, 14, 14
HW = H * W
ROWS_PER_STEP = 8


def _gather_mask_plane(indices, table_i8):
    def body(idx_ref, *refs):
        tables = refs[:ROWS_PER_STEP]
        o_ref = refs[ROWS_PER_STEP]
        i = pl.program_id(0)
        row_iota = jax.lax.broadcasted_iota(jnp.int32, (8, C), 0)
        for j in range(ROWS_PER_STEP):
            r = idx_ref[i * ROWS_PER_STEP + j] % 8
            slab = tables[j][...].astype(jnp.float32)
            row = jnp.sum(jnp.where(row_iota == r, slab, 0.0), axis=0,
                          keepdims=True)
            o_ref[pl.ds(j, 1), :] = row

    def make_spec(j):
        return pl.BlockSpec(
            (8, C), lambda i, idx, j=j: (idx[i * ROWS_PER_STEP + j] // 8, 0)
        )

    grid_spec = pltpu.PrefetchScalarGridSpec(
        num_scalar_prefetch=1,
        grid=(B // ROWS_PER_STEP,),
        in_specs=[make_spec(j) for j in range(ROWS_PER_STEP)],
        out_specs=pl.BlockSpec((ROWS_PER_STEP, C), lambda i, idx: (i, 0)),
    )
    return pl.pallas_call(
        body,
        grid_spec=grid_spec,
        out_shape=jax.ShapeDtypeStruct((B, C), jnp.float32),
    )(indices, *([table_i8] * ROWS_PER_STEP))


def _mask_multiply(x_planes, mask):
    p = 14
    grid = (HW // p,)

    def body(x_ref, m_ref, o_ref):
        o_ref[...] = x_ref[...] * m_ref[...][None, :, :]

    return pl.pallas_call(
        body,
        grid=grid,
        in_specs=[
            pl.BlockSpec((p, B, C), lambda i: (i, 0, 0)),
            pl.BlockSpec((B, C), lambda i: (0, 0)),
        ],
        out_specs=pl.BlockSpec((p, B, C), lambda i: (i, 0, 0)),
        out_shape=jax.ShapeDtypeStruct((HW, B, C), jnp.float32),
    )(x_planes, mask)


def kernel(X, indices, mask_table):
    table_i8 = mask_table.view(jnp.int8)
    mask = _gather_mask_plane(indices, table_i8)
    x_planes = jnp.transpose(X, (2, 3, 0, 1)).reshape(HW, B, C)
    out = _mask_multiply(x_planes, mask)
    return jnp.transpose(out.reshape(H, W, B, C), (2, 3, 0, 1))


# speedup vs baseline: 5.3367x; 1.0222x over previous
import jax
import jax.numpy as jnp
from jax.experimental import pallas as pl
from jax.experimental.pallas import tpu as pltpu

B, C, H, W = 256, 256, 14, 14
HW = H * W
ROWS_PER_STEP = 8


def _gather_mask_plane(indices, table_i8):
    def body(idx_ref, *refs):
        tables = refs[:ROWS_PER_STEP]
        o_ref = refs[ROWS_PER_STEP]
        i = pl.program_id(0)
        row_iota = jax.lax.broadcasted_iota(jnp.int32, (8, C), 0)
        for j in range(ROWS_PER_STEP):
            r = idx_ref[i * ROWS_PER_STEP + j] % 8
            slab = tables[j][...].astype(jnp.float32)
            row = jnp.sum(jnp.where(row_iota == r, slab, 0.0), axis=0,
                          keepdims=True)
            o_ref[pl.ds(j, 1), :] = row

    def make_spec(j):
        return pl.BlockSpec(
            (8, C), lambda i, idx, j=j: (idx[i * ROWS_PER_STEP + j] // 8, 0)
        )

    grid_spec = pltpu.PrefetchScalarGridSpec(
        num_scalar_prefetch=1,
        grid=(B // ROWS_PER_STEP,),
        in_specs=[make_spec(j) for j in range(ROWS_PER_STEP)],
        out_specs=pl.BlockSpec((ROWS_PER_STEP, C), lambda i, idx: (i, 0)),
    )
    return pl.pallas_call(
        body,
        grid_spec=grid_spec,
        out_shape=jax.ShapeDtypeStruct((B, C), jnp.float32),
    )(indices, *([table_i8] * ROWS_PER_STEP))


def _mask_multiply(x_planes, mask):
    p = 28
    grid = (HW // p,)

    def body(x_ref, m_ref, o_ref):
        o_ref[...] = x_ref[...] * m_ref[...][None, :, :]

    return pl.pallas_call(
        body,
        grid=grid,
        in_specs=[
            pl.BlockSpec((p, B, C), lambda i: (i, 0, 0)),
            pl.BlockSpec((B, C), lambda i: (0, 0)),
        ],
        out_specs=pl.BlockSpec((p, B, C), lambda i: (i, 0, 0)),
        out_shape=jax.ShapeDtypeStruct((HW, B, C), jnp.float32),
    )(x_planes, mask)


def kernel(X, indices, mask_table):
    table_i8 = mask_table.view(jnp.int8)
    mask = _gather_mask_plane(indices, table_i8)
    x_planes = jnp.transpose(X, (2, 3, 0, 1)).reshape(HW, B, C)
    out = _mask_multiply(x_planes, mask)
    return jnp.transpose(out.reshape(H, W, B, C), (2, 3, 0, 1))
